# baseline (speedup 1.0000x reference)
import jax
import jax.numpy as jnp
from jax import lax
from jax.experimental import pallas as pl
from jax.experimental.pallas import tpu as pltpu
from jax.experimental.pallas import tpu_sc as plsc

_S = 7
_NCH = 30
_NCHP = 32
_BATCH = 128
_P = _S * _S
_NW = 16
_PPW = 4
_POS_F = _NCHP * _BATCH
_FPW = _PPW * _POS_F
_NG = _BATCH // 16
_L_COORD = 5.0
_L_NOOBJ = 0.5


def _sqrt16(x):
    i = plsc.bitcast(x, jnp.int32)
    i = jnp.int32(0x5F3759DF) - jnp.right_shift(i, 1)
    z = plsc.bitcast(i, jnp.float32)
    xh = 0.5 * x
    for _ in range(2):
        z = z * (1.5 - xh * z * z)
    return x * z


def _corners(cx, cy, w, h):
    x = cx * (1.0 / _S)
    y = cy * (1.0 / _S)
    return x - 0.5 * w, y - 0.5 * h, x + 0.5 * w, y + 0.5 * h


def _sc_body(pred_hbm, tgt_hbm, out_hbm, pred_v, tgt_v, acc_v, sem_p, sem_t):
    wid = lax.axis_index("s")
    k_lo = (_P * _NG * wid) // _NW
    k_hi = (_P * _NG * (wid + 1)) // _NW
    base_p = k_lo // _NG
    cp = pltpu.async_copy(
        pred_hbm.at[pl.ds(base_p * _POS_F, _FPW)], pred_v, sem_p)
    ct = pltpu.async_copy(
        tgt_hbm.at[pl.ds(base_p * _POS_F, _FPW)], tgt_v, sem_t)
    cp.wait()
    ct.wait()

    def unit_loss(k, acc):
            p = k // _NG
            g = k - p * _NG
            off = (p - base_p) * _POS_F + g * 16

            def gp(c):
                return pred_v[pl.ds(off + c * _BATCH, 16)]

            def gt(c):
                return tgt_v[pl.ds(off + c * _BATCH, 16)]

            t_x, t_y, t_w, t_h, t_conf = gt(0), gt(1), gt(2), gt(3), gt(4)
            tx1, ty1, tx2, ty2 = _corners(t_x, t_y, t_w, t_h)
            area_t = (tx2 - tx1) * (ty2 - ty1)

            def iou_parts(px, py, pw, ph):
                x1, y1, x2, y2 = _corners(px, py, pw, ph)
                iw = jnp.maximum(
                    jnp.minimum(x2, tx2) - jnp.maximum(x1, tx1), 0.0)
                ih = jnp.maximum(
                    jnp.minimum(y2, ty2) - jnp.maximum(y1, ty1), 0.0)
                inter = iw * ih
                area_p = (x2 - x1) * (y2 - y1)
                return inter, area_p + area_t - inter

            p0b = [gp(c) for c in range(5)]
            p1b = [gp(c) for c in range(5, 10)]
            in0, de0 = iou_parts(p0b[0], p0b[1], p0b[2], p0b[3])
            in1, de1 = iou_parts(p1b[0], p1b[1], p1b[2], p1b[3])
            sel = in1 * de0 > in0 * de1
            max_iou = jnp.where(sel, in1, in0) / jnp.where(sel, de1, de0)

            r = [jnp.where(sel, b1, b0) for b0, b1 in zip(p0b, p1b)]
            t1 = [gt(c) for c in range(5, 9)]
            tr = [jnp.where(sel, b1, b0)
                  for b0, b1 in zip((t_x, t_y, t_w, t_h), t1)]

            dx = r[0] - tr[0]
            dy = r[1] - tr[1]
            l_xy = dx * dx + dy * dy
            l_wh = (r[2] + tr[2] - 2.0 * _sqrt16(r[2] * tr[2])
                    + r[3] + tr[3] - 2.0 * _sqrt16(r[3] * tr[3]))
            do = r[4] - max_iou
            l_obj = do * do

            dn0 = p0b[4] - t_conf
            dn1 = p1b[4] - gt(9)
            l_noobj = dn0 * dn0 + dn1 * dn1

            l_cls = jnp.zeros((16,), jnp.float32)
            for c in range(10, 30):
                d = gp(c) - gt(c)
                l_cls = l_cls + d * d

            obj_f = jnp.where(t_conf > 0.0, 1.0, 0.0)
            noobj_f = jnp.where(t_conf == 0.0, 1.0, 0.0)
            cell = (obj_f * (_L_COORD * (l_xy + l_wh) + l_obj + l_cls)
                    + _L_NOOBJ * noobj_f * l_noobj)
            return acc + cell

    acc = lax.fori_loop(k_lo, k_hi, unit_loss,
                        jnp.zeros((16,), jnp.float32))
    acc_v[...] = acc
    pltpu.sync_copy(acc_v, out_hbm.at[wid])


@jax.jit
def kernel(pred_tensor, target_tensor):
    def prep(x):
        x = jnp.transpose(x, (1, 2, 3, 0)).reshape(_P, _NCH, _BATCH)
        z = jnp.zeros((_P, _NCHP - _NCH, _BATCH), jnp.float32)
        return jnp.concatenate([x, z], axis=1).reshape(_P * _POS_F)

    partials = pl.kernel(
        _sc_body,
        out_type=jax.ShapeDtypeStruct((_NW, 16), jnp.float32),
        mesh=plsc.VectorSubcoreMesh(core_axis_name="c", subcore_axis_name="s",
                                    num_cores=1, num_subcores=16),
        scratch_types=[
            pltpu.VMEM((_FPW,), jnp.float32),
            pltpu.VMEM((_FPW,), jnp.float32),
            pltpu.VMEM((16,), jnp.float32),
            pltpu.SemaphoreType.DMA,
            pltpu.SemaphoreType.DMA,
        ],
        compiler_params=pltpu.CompilerParams(needs_layout_passes=False),
    )(prep(pred_tensor), prep(target_tensor))
    return jnp.sum(partials) / float(_BATCH)
